# Initial kernel scaffold; baseline (speedup 1.0000x reference)
import jax
import jax.numpy as jnp
from jax.experimental import pallas as pl


def kernel(coords, atom_description, alternativeMask, facc, weight, atom_Properties):
    raise NotImplementedError("write your pallas kernel here")


# speedup vs baseline: 3.0913x; 3.0913x over previous
import functools

import jax
import jax.numpy as jnp
from jax import lax
from jax.experimental import pallas as pl
from jax.experimental.pallas import tpu as pltpu
from jax.experimental.pallas import tpu_sc as plsc

N_ATOMS = 500000
NALTERN = 4
N_PROPS = 8
NBINS = 8 * 4 * 512
ACC_WORDS = NBINS * NALTERN
N_SUBCORES = 16
N_CORES = 2
PAD_ATOMS = 512000
ATOMS_PER_SUB = PAD_ATOMS // N_SUBCORES
CHUNK_ATOMS = 2000
N_CHUNKS = ATOMS_PER_SUB // CHUNK_ATOMS
STEPS = CHUNK_ATOMS // 16
CHUNK_WORDS = CHUNK_ATOMS * 4


def _sc_partials_kernel(desc_hbm, facc_hbm, mask_hbm, props_hbm, out_hbm,
                        desc_v, facc_v, mask_v, props_v, acc_v):
    c = lax.axis_index("c")
    s = lax.axis_index("s")

    pltpu.sync_copy(props_hbm, props_v)

    zeros16 = jnp.zeros((16,), jnp.float32)

    def zero_body(i, carry):
        acc_v[pl.ds(i * 16, 16)] = zeros16
        return carry

    lax.fori_loop(0, ACC_WORDS // 16, zero_body, 0)

    lane = lax.iota(jnp.int32, 16)
    lane4 = lane * 4
    cvec = jnp.broadcast_to(c, (16,)).astype(jnp.int32)
    pair_base = s * ATOMS_PER_SUB

    def chunk_body(k, carry):
        base = (pair_base + k * CHUNK_ATOMS) * 4
        pltpu.sync_copy(desc_hbm.at[pl.ds(base, CHUNK_WORDS)], desc_v)
        pltpu.sync_copy(facc_hbm.at[pl.ds(base, CHUNK_WORDS)], facc_v)
        pltpu.sync_copy(mask_hbm.at[pl.ds(base, CHUNK_WORDS)], mask_v)

        def step_body(t, carry2):
            o = lane4 + t * 64
            b = plsc.load_gather(desc_v, [o])
            ch = plsc.load_gather(desc_v, [o + 1])
            r = plsc.load_gather(desc_v, [o + 2])
            at = plsc.load_gather(desc_v, [o + 3])
            vdw = plsc.load_gather(props_v, [at * N_PROPS])
            sel = (at >= 4).astype(jnp.int32) == cvec
            binidx = b * 8192 + ch * 2048 + r * 4
            for alt in range(NALTERN):
                fa = plsc.load_gather(facc_v, [o + alt])
                mf = plsc.load_gather(mask_v, [o + alt])
                val = jnp.maximum(fa, 0.0) * vdw * mf
                plsc.addupdate_scatter(acc_v, [binidx + alt], val, mask=sel)
            return carry2

        lax.fori_loop(0, STEPS, step_body, 0)
        return carry

    lax.fori_loop(0, N_CHUNKS, chunk_body, 0)

    pltpu.sync_copy(acc_v, out_hbm.at[c, s])


_sc_partials = functools.partial(
    pl.kernel,
    out_type=jax.ShapeDtypeStruct((N_CORES, N_SUBCORES, ACC_WORDS), jnp.float32),
    mesh=plsc.VectorSubcoreMesh(core_axis_name="c", subcore_axis_name="s"),
    compiler_params=pltpu.CompilerParams(needs_layout_passes=False),
    scratch_types=[
        pltpu.VMEM((CHUNK_WORDS,), jnp.int32),
        pltpu.VMEM((CHUNK_WORDS,), jnp.float32),
        pltpu.VMEM((CHUNK_WORDS,), jnp.float32),
        pltpu.VMEM((N_PROPS * 64,), jnp.float32),
        pltpu.VMEM((ACC_WORDS,), jnp.float32),
    ],
)(_sc_partials_kernel)


def _tc_reduce_kernel(p_ref, w_ref, out_ref):
    scale = (1.0 - jnp.tanh(w_ref[0, 0])) * 0.3
    out_ref[...] = jnp.sum(p_ref[...], axis=1) * scale


def _tc_reduce(partials, weight):
    cols = ACC_WORDS // 8
    return pl.pallas_call(
        _tc_reduce_kernel,
        grid=(8,),
        in_specs=[
            pl.BlockSpec((N_CORES, N_SUBCORES, cols), lambda j: (0, 0, j)),
            pl.BlockSpec(memory_space=pltpu.SMEM),
        ],
        out_specs=pl.BlockSpec((N_CORES, cols), lambda j: (0, j)),
        out_shape=jax.ShapeDtypeStruct((N_CORES, ACC_WORDS), jnp.float32),
    )(partials, weight)


@jax.jit
def kernel(coords, atom_description, alternativeMask, facc, weight, atom_Properties):
    del coords
    pad = PAD_ATOMS - N_ATOMS
    desc_f = jnp.pad(atom_description, ((0, pad), (0, 0))).reshape(-1)
    facc_f = jnp.pad(facc, ((0, pad), (0, 0))).reshape(-1)
    mask_f = jnp.pad(alternativeMask.astype(jnp.float32), ((0, pad), (0, 0))).reshape(-1)
    props_f = jnp.pad(atom_Properties.astype(jnp.float32).reshape(-1),
                      (0, N_PROPS * 64 - N_PROPS * atom_Properties.shape[0]))

    partials = _sc_partials(desc_f, facc_f, mask_f, props_f)
    out2 = _tc_reduce(partials, weight.reshape(1, 1))
    final_mc = out2[0].reshape(8, 4, 512, NALTERN)
    final_sc = out2[1].reshape(8, 4, 512, NALTERN)
    return (final_mc, final_sc)


# speedup vs baseline: 4.9949x; 1.6158x over previous
import functools

import jax
import jax.numpy as jnp
from jax import lax
from jax.experimental import pallas as pl
from jax.experimental.pallas import tpu as pltpu
from jax.experimental.pallas import tpu_sc as plsc

N_ATOMS = 500000
NALTERN = 4
NBINS = 8 * 4 * 512
ACC_WORDS = NBINS * NALTERN
N_SUBCORES = 16
N_CORES = 2
CHUNK = 6256
N_CHUNKS = 80
CHUNKS_PER_SUB = N_CHUNKS // N_SUBCORES
STEPS = CHUNK // 16
TAIL_BASE = N_ATOMS - CHUNK
TAIL_PREFIX = 79 * CHUNK - TAIL_BASE


def _sc_partials_kernel(desc_hbm, facc_hbm, mask_hbm, props_hbm, out_hbm,
                        desc_v, facc_v, mask_v, props_v, acc_v):
    c = lax.axis_index("c")
    s = lax.axis_index("s")

    pltpu.sync_copy(props_hbm, props_v)

    zeros16 = jnp.zeros((16,), jnp.float32)

    def zero_body(i, carry):
        acc_v[pl.ds(i * 16, 16)] = zeros16
        return carry

    lax.fori_loop(0, ACC_WORDS // 16, zero_body, 0)

    lane = lax.iota(jnp.int32, 16)
    col0 = lane * 0
    cvec = jnp.broadcast_to(c, (16,)).astype(jnp.int32)

    def chunk_body(k2, carry):
        kk = s * CHUNKS_PER_SUB + k2
        is_tail = kk >= N_CHUNKS - 1
        base = jnp.where(is_tail, TAIL_BASE, kk * CHUNK)
        prefix = jnp.where(is_tail, TAIL_PREFIX, 0)
        prefix_v = jnp.broadcast_to(prefix, (16,))
        pltpu.sync_copy(desc_hbm.at[pl.ds(base * 4, CHUNK * 4)], desc_v)
        pltpu.sync_copy(facc_hbm.at[pl.ds(base * 4, CHUNK * 4)], facc_v)
        pltpu.sync_copy(mask_hbm.at[pl.ds(base, CHUNK)], mask_v)

        def step_body(t, carry2):
            row = lane + t * 16
            o = row * 4
            b = plsc.load_gather(desc_v, [o])
            ch = plsc.load_gather(desc_v, [o + 1])
            r = plsc.load_gather(desc_v, [o + 2])
            at = plsc.load_gather(desc_v, [o + 3])
            vdw = plsc.load_gather(props_v, [at * 8])
            mword = plsc.load_gather(mask_v, [row])
            sel = ((at >= 4).astype(jnp.int32) == cvec) & (row >= prefix_v)
            binidx = b * 8192 + ch * 2048 + r * 4
            for alt in range(NALTERN):
                fa = plsc.load_gather(facc_v, [o + alt])
                mf = ((mword >> (8 * alt)) & 1).astype(jnp.float32)
                val = jnp.maximum(fa, 0.0) * vdw * mf
                plsc.addupdate_scatter(acc_v, [binidx + alt], val, mask=sel)
            return carry2

        lax.fori_loop(0, STEPS, step_body, 0)
        return carry

    lax.fori_loop(0, CHUNKS_PER_SUB, chunk_body, 0)

    pltpu.sync_copy(acc_v, out_hbm.at[c, s])


_sc_partials = functools.partial(
    pl.kernel,
    out_type=jax.ShapeDtypeStruct((N_CORES, N_SUBCORES, ACC_WORDS), jnp.float32),
    mesh=plsc.VectorSubcoreMesh(core_axis_name="c", subcore_axis_name="s"),
    compiler_params=pltpu.CompilerParams(needs_layout_passes=False),
    scratch_types=[
        pltpu.VMEM((CHUNK * 4,), jnp.int32),
        pltpu.VMEM((CHUNK * 4,), jnp.float32),
        pltpu.VMEM((CHUNK,), jnp.int32),
        pltpu.VMEM((320,), jnp.float32),
        pltpu.VMEM((ACC_WORDS,), jnp.float32),
    ],
)(_sc_partials_kernel)


def _tc_reduce_kernel(p_ref, w_ref, out_ref):
    scale = (1.0 - jnp.tanh(w_ref[0, 0])) * 0.3
    out_ref[...] = jnp.sum(p_ref[...], axis=1) * scale


def _tc_reduce(partials, weight):
    cols = ACC_WORDS // 8
    return pl.pallas_call(
        _tc_reduce_kernel,
        grid=(8,),
        in_specs=[
            pl.BlockSpec((N_CORES, N_SUBCORES, cols), lambda j: (0, 0, j)),
            pl.BlockSpec(memory_space=pltpu.SMEM),
        ],
        out_specs=pl.BlockSpec((N_CORES, cols), lambda j: (0, j)),
        out_shape=jax.ShapeDtypeStruct((N_CORES, ACC_WORDS), jnp.float32),
    )(partials, weight)


@jax.jit
def kernel(coords, atom_description, alternativeMask, facc, weight, atom_Properties):
    del coords
    mask_packed = lax.bitcast_convert_type(
        alternativeMask.astype(jnp.int8), jnp.int32)
    partials = _sc_partials(atom_description.reshape(-1), facc.reshape(-1),
                            mask_packed, atom_Properties.reshape(-1))
    out2 = _tc_reduce(partials, weight.reshape(1, 1))
    final_mc = out2[0].reshape(8, 4, 512, NALTERN)
    final_sc = out2[1].reshape(8, 4, 512, NALTERN)
    return (final_mc, final_sc)
